# baseline (speedup 1.0000x reference)
import jax
import jax.numpy as jnp
from jax.experimental import pallas as pl

MAX_H = 64
MAX_W = 64


def _add_body(x_ref, pt_ref, o_ref, *, H, WC):
    o_ref[...] = x_ref[...] + pt_ref[:H, :WC][None, :, :]


def kernel(x, pos_table):
    B, H, W, C = x.shape
    WC = W * C
    x_r = x.reshape(B, H, WC)
    pt_r = pos_table.reshape(MAX_H, MAX_W * C)

    BB = 8

    out = pl.pallas_call(
        lambda x_ref, pt_ref, o_ref: _add_body(x_ref, pt_ref, o_ref, H=H, WC=WC),
        grid=(B // BB,),
        in_specs=[
            pl.BlockSpec((BB, H, WC), lambda i: (i, 0, 0)),
            pl.BlockSpec((MAX_H, MAX_W * C), lambda i: (0, 0)),
        ],
        out_specs=pl.BlockSpec((BB, H, WC), lambda i: (i, 0, 0)),
        out_shape=jax.ShapeDtypeStruct((B, H, WC), x.dtype),
    )(x_r, pt_r)
    return out.reshape(B, H, W, C)
